# Initial kernel scaffold; baseline (speedup 1.0000x reference)
import jax
import jax.numpy as jnp
from jax.experimental import pallas as pl


def kernel(x, edge_index, Wc_out, bc_out, Wc_root, Wg1, a_src1, a_dst1, bg1, Wg2, a_src2, a_dst2, bg2):
    raise NotImplementedError("write your pallas kernel here")


# speedup vs baseline: 20.1598x; 20.1598x over previous
import functools

import jax
import jax.numpy as jnp
from jax import lax
from jax.experimental import pallas as pl
from jax.experimental.pallas import tpu as pltpu
from jax.experimental.pallas import tpu_sc as plsc

N = 10000
D = 128
E = 320000
E2 = N + E
NC = 2
NS = 16
TILES = NC * NS
G = 128
C = -(-E2 // (TILES * G))
PT = C * G
EP = TILES * PT
NT = 10240
DUMMY = N
NV = NT // 16
RPT = NT // NS
NG = NT // G
DV = D // 16
BR = 256

_f32 = jnp.float32
_i32 = jnp.int32


def _iota16():
  return lax.iota(_i32, 16)


def _take(v, idx):
  return v.at[idx].get(mode="promise_in_bounds")


def _seg_rmw(tab_ref, keys, vals, *, is_max):
  ks, vs = plsc.sort_key_val(keys, vals)
  ii = _iota16()
  for st in (1, 2, 4, 8):
    idx = jnp.minimum(ii + st, 15)
    nk = _take(ks, idx)
    nv = _take(vs, idx)
    ok = (ii + st <= 15) & (nk == ks)
    comb = jnp.maximum(vs, nv) if is_max else vs + nv
    vs = jnp.where(ok, comb, vs)
  pk = _take(ks, jnp.maximum(ii - 1, 0))
  first = (ks != pk) | (ii == 0)
  old = plsc.load_gather(tab_ref, [ks])
  comb = jnp.maximum(old, vs) if is_max else old + vs
  plsc.store_scatter(tab_ref, [ks], comb, mask=first)


def _fill(ref, nvec, val):
  v = jnp.full((16,), val, _f32)

  def body(i, carry):
    ref[pl.ds(i * 16, 16)] = v
    return carry

  lax.fori_loop(0, nvec, body, 0)


def _combine_sum(tab_ref, shared_ref, idn_ref, zv_ref, out_row, s):
  _fill(zv_ref, RPT // 16, 0.0)
  pltpu.sync_copy(zv_ref, shared_ref.at[pl.ds(s * RPT, RPT)])
  plsc.subcore_barrier()

  def body(q, carry):
    pltpu.sync_copy(tab_ref.at[pl.ds(q * G, G)],
                    shared_ref.at[idn_ref.at[q]], add=True)
    return carry

  lax.fori_loop(0, NG, body, 0)
  plsc.subcore_barrier()
  pltpu.sync_copy(shared_ref.at[pl.ds(s * RPT, RPT)],
                  out_row.at[pl.ds(s * RPT, RPT)])


def _combine_max(tab_ref, stage_ref, red_ref, outsm_ref, out_row, s):
  pltpu.sync_copy(tab_ref, stage_ref.at[s])
  plsc.subcore_barrier()
  for r in range(NS):
    pltpu.sync_copy(stage_ref.at[r, pl.ds(s * RPT, RPT)], red_ref.at[r])

  def body(q, carry):
    acc = red_ref[0, pl.ds(q * 16, 16)]
    for r in range(1, NS):
      acc = jnp.maximum(acc, red_ref[r, pl.ds(q * 16, 16)])
    outsm_ref[pl.ds(q * 16, 16)] = acc
    return carry

  lax.fori_loop(0, RPT // 16, body, 0)
  pltpu.sync_copy(outsm_ref, out_row.at[pl.ds(s * RPT, RPT)])


def _deg_body(srcp, dstp, idn, deg2, dst_r,
              src_v, dst_v, idn_v, tab, zv, shared):
  c = lax.axis_index("c")
  s = lax.axis_index("s")
  t = c * NS + s
  pltpu.sync_copy(srcp.at[t], src_v)
  pltpu.sync_copy(dstp.at[t], dst_v)
  pltpu.sync_copy(idn, idn_v)
  _fill(tab, NV, 0.0)
  base = t * PT

  def chunk(j, carry):
    for k in range(G // 16):
      sv = src_v[j, pl.ds(k * 16, 16)]
      dv = dst_v[j, pl.ds(k * 16, 16)]
      pos = base + j * G + k * 16 + _iota16()
      real = sv != dv
      validf = jnp.where(real | (pos >= E), 1.0, 0.0).astype(_f32)
      _seg_rmw(tab, dv, validf, is_max=False)
      keep = real & (pos < E)
      dst_v[j, pl.ds(k * 16, 16)] = jnp.where(keep, dv, DUMMY)
    return carry

  lax.fori_loop(0, C, chunk, 0)
  pltpu.sync_copy(dst_v, dst_r.at[t])
  _combine_sum(tab, shared, idn_v, zv, deg2.at[c], s)


def _gat_a_body(srcp, dstp, s_hbm, d_hbm, e_out, m2,
                src_v, dst_v, s_tab, d_tab, m_tab, e_v, stage, red, outsm):
  c = lax.axis_index("c")
  s = lax.axis_index("s")
  t = c * NS + s
  pltpu.sync_copy(srcp.at[t], src_v)
  pltpu.sync_copy(dstp.at[t], dst_v)
  pltpu.sync_copy(s_hbm, s_tab)
  pltpu.sync_copy(d_hbm, d_tab)
  _fill(m_tab, NV, -1e30)
  base = t * PT

  def chunk(j, carry):
    for k in range(G // 16):
      sv = src_v[j, pl.ds(k * 16, 16)]
      dv = dst_v[j, pl.ds(k * 16, 16)]
      pos = base + j * G + k * 16 + _iota16()
      ssv = plsc.load_gather(s_tab, [sv])
      ddv = plsc.load_gather(d_tab, [dv])
      z = ssv + ddv
      e16 = jnp.where(z > 0, z, 0.2 * z)
      vb = (sv != dv) | (pos >= E)
      e16 = jnp.where(vb, e16, -1e30)
      e_v[j, pl.ds(k * 16, 16)] = e16
      _seg_rmw(m_tab, dv, e16, is_max=True)
    return carry

  lax.fori_loop(0, C, chunk, 0)
  pltpu.sync_copy(e_v, e_out.at[t])
  _combine_max(m_tab, stage, red, outsm, m2.at[c], s)


def _gat_b_body(dstp, e_in, m2, idn, ee_out, den2,
                dst_v, idn_v, m_tab, tmp_tab, den_tab, e_v, ee_v, zv,
                shared):
  c = lax.axis_index("c")
  s = lax.axis_index("s")
  t = c * NS + s
  pltpu.sync_copy(dstp.at[t], dst_v)
  pltpu.sync_copy(e_in.at[t], e_v)
  pltpu.sync_copy(idn, idn_v)
  pltpu.sync_copy(m2.at[0], m_tab)
  pltpu.sync_copy(m2.at[1], tmp_tab)

  def mcomb(q, carry):
    m_tab[pl.ds(q * 16, 16)] = jnp.maximum(
        m_tab[pl.ds(q * 16, 16)], tmp_tab[pl.ds(q * 16, 16)])
    return carry

  lax.fori_loop(0, NV, mcomb, 0)
  _fill(den_tab, NV, 0.0)

  def chunk(j, carry):
    for k in range(G // 16):
      dv = dst_v[j, pl.ds(k * 16, 16)]
      ev = e_v[j, pl.ds(k * 16, 16)]
      mv = plsc.load_gather(m_tab, [dv])
      eev = jnp.exp(ev - mv)
      ee_v[j, pl.ds(k * 16, 16)] = eev
      _seg_rmw(den_tab, dv, eev, is_max=False)
    return carry

  lax.fori_loop(0, C, chunk, 0)
  pltpu.sync_copy(ee_v, ee_out.at[t])
  _combine_sum(den_tab, shared, idn_v, zv, den2.at[c], s)


def _spmm_body(table, srcp, dstp, w_hbm, acc2,
               src_v, dst_v, wbuf, rowbuf, acc_sh, gsem):
  c = lax.axis_index("c")
  s = lax.axis_index("s")
  t = c * NS + s
  pltpu.sync_copy(srcp.at[t], src_v)
  pltpu.sync_copy(dstp.at[t], dst_v)

  def zrow(r, carry):
    for v in range(DV):
      rowbuf[r, pl.ds(v * 16, 16)] = jnp.zeros((16,), _f32)
    return carry

  lax.fori_loop(0, G, zrow, 0)
  for i in range(RPT // G):
    pltpu.sync_copy(rowbuf, acc_sh.at[pl.ds(s * RPT + i * G, G)])
  plsc.subcore_barrier()

  def chunk(j, carry):
    pltpu.sync_copy(w_hbm.at[t, j], wbuf)
    pltpu.async_copy(table.at[src_v.at[j]], rowbuf, gsem).wait()

    def erow(g, icarry):
      w16 = wbuf[pl.ds(g * 16, 16)]
      for l in range(16):
        w = w16[l]
        r = g * 16 + l
        for v in range(DV):
          rowbuf[r, pl.ds(v * 16, 16)] = rowbuf[r, pl.ds(v * 16, 16)] * w
      return icarry

    lax.fori_loop(0, G // 16, erow, 0)
    pltpu.sync_copy(rowbuf, acc_sh.at[dst_v.at[j]], add=True)
    return carry

  lax.fori_loop(0, C, chunk, 0)
  plsc.subcore_barrier()

  for i in range(RPT // G):
    lo = s * RPT + i * G
    pltpu.sync_copy(acc_sh.at[pl.ds(lo, G)], rowbuf)
    pltpu.sync_copy(rowbuf, acc2.at[c, pl.ds(lo, G)])


@functools.cache
def _sc_kernels():
  mesh = plsc.VectorSubcoreMesh(
      core_axis_name="c", subcore_axis_name="s",
      num_cores=NC, num_subcores=NS)
  cparams = pltpu.CompilerParams(needs_layout_passes=False)
  k_deg = pl.kernel(
      _deg_body,
      out_type=(
          jax.ShapeDtypeStruct((NC, NT), _f32),
          jax.ShapeDtypeStruct((TILES, C, G), _i32),
      ),
      mesh=mesh,
      compiler_params=cparams,
      scratch_types=[
          pltpu.VMEM((C, G), _i32),
          pltpu.VMEM((C, G), _i32),
          pltpu.VMEM((NG, G), _i32),
          pltpu.VMEM((NT,), _f32),
          pltpu.VMEM((RPT,), _f32),
          pltpu.VMEM_SHARED((NT,), _f32),
      ],
  )
  k_gat_a = pl.kernel(
      _gat_a_body,
      out_type=(
          jax.ShapeDtypeStruct((TILES, C, G), _f32),
          jax.ShapeDtypeStruct((NC, NT), _f32),
      ),
      mesh=mesh,
      compiler_params=cparams,
      scratch_types=[
          pltpu.VMEM((C, G), _i32),
          pltpu.VMEM((C, G), _i32),
          pltpu.VMEM((NT,), _f32),
          pltpu.VMEM((NT,), _f32),
          pltpu.VMEM((NT,), _f32),
          pltpu.VMEM((C, G), _f32),
          pltpu.VMEM_SHARED((NS, NT), _f32),
          pltpu.VMEM((NS, RPT), _f32),
          pltpu.VMEM((RPT,), _f32),
      ],
  )
  k_gat_b = pl.kernel(
      _gat_b_body,
      out_type=(
          jax.ShapeDtypeStruct((TILES, C, G), _f32),
          jax.ShapeDtypeStruct((NC, NT), _f32),
      ),
      mesh=mesh,
      compiler_params=cparams,
      scratch_types=[
          pltpu.VMEM((C, G), _i32),
          pltpu.VMEM((NG, G), _i32),
          pltpu.VMEM((NT,), _f32),
          pltpu.VMEM((NT,), _f32),
          pltpu.VMEM((NT,), _f32),
          pltpu.VMEM((C, G), _f32),
          pltpu.VMEM((C, G), _f32),
          pltpu.VMEM((RPT,), _f32),
          pltpu.VMEM_SHARED((NT,), _f32),
      ],
  )
  k_spmm = pl.kernel(
      _spmm_body,
      out_type=jax.ShapeDtypeStruct((NC, NT, D), _f32),
      mesh=mesh,
      compiler_params=cparams,
      scratch_types=[
          pltpu.VMEM((C, G), _i32),
          pltpu.VMEM((C, G), _i32),
          pltpu.VMEM((G,), _f32),
          pltpu.VMEM((G, D), _f32),
          pltpu.VMEM_SHARED((NT, D), _f32),
          pltpu.SemaphoreType.DMA,
      ],
  )
  return k_deg, k_gat_a, k_gat_b, k_spmm


def _blk_rows(i):
  return (i, 0)


_full = pl.BlockSpec((D, D), lambda i: (0, 0))
_vec = pl.BlockSpec((1, D), lambda i: (0, 0))


def _tca_body(acc_r, deg_r, x_r, wot_r, wrt_r, bo_r, w1t_r, as2_r, ad2_r,
              hw_r, sd_r):
  dinv = 1.0 / jnp.maximum(deg_r[0] + deg_r[1], 1.0)
  a = (acc_r[0] + acc_r[1] + 2.0 * x_r[...]) * dinv
  h = jnp.dot(a, wot_r[...]) + jnp.dot(x_r[...], wrt_r[...]) + bo_r[0]
  h = jnp.maximum(h, 0.0)
  hw_r[...] = jnp.dot(h, w1t_r[...])
  sd_r[0] = h @ as2_r[0]
  sd_r[1] = h @ ad2_r[0]


tc_a = pl.pallas_call(
    _tca_body,
    grid=(NT // BR,),
    in_specs=[
        pl.BlockSpec((NC, BR, D), lambda i: (0, i, 0)),
        pl.BlockSpec((NC, BR, 1), lambda i: (0, i, 0)),
        pl.BlockSpec((BR, D), _blk_rows),
        _full, _full, _vec, _full, _vec, _vec,
    ],
    out_specs=[
        pl.BlockSpec((BR, D), _blk_rows),
        pl.BlockSpec((2, BR), lambda i: (0, i)),
    ],
    out_shape=[
        jax.ShapeDtypeStruct((NT, D), _f32),
        jax.ShapeDtypeStruct((2, NT), _f32),
    ],
)


def _tcb_body(acc_r, den_r, b_r, w2t_r, as2_r, ad2_r, hw_r, sd_r):
  dscl = 1.0 / (den_r[0] + den_r[1] + 1e-16)
  h = jnp.maximum((acc_r[0] + acc_r[1]) * dscl + b_r[0], 0.0)
  hw_r[...] = jnp.dot(h, w2t_r[...])
  sd_r[0] = h @ as2_r[0]
  sd_r[1] = h @ ad2_r[0]


tc_b = pl.pallas_call(
    _tcb_body,
    grid=(NT // BR,),
    in_specs=[
        pl.BlockSpec((NC, BR, D), lambda i: (0, i, 0)),
        pl.BlockSpec((NC, BR, 1), lambda i: (0, i, 0)),
        _vec, _full, _vec, _vec,
    ],
    out_specs=[
        pl.BlockSpec((BR, D), _blk_rows),
        pl.BlockSpec((2, BR), lambda i: (0, i)),
    ],
    out_shape=[
        jax.ShapeDtypeStruct((NT, D), _f32),
        jax.ShapeDtypeStruct((2, NT), _f32),
    ],
)


def _tcc_body(acc_r, den_r, b_r, out_r):
  dscl = 1.0 / (den_r[0] + den_r[1] + 1e-16)
  out_r[...] = (acc_r[0] + acc_r[1]) * dscl + b_r[0]


tc_c = pl.pallas_call(
    _tcc_body,
    grid=(NT // BR,),
    in_specs=[
        pl.BlockSpec((NC, BR, D), lambda i: (0, i, 0)),
        pl.BlockSpec((NC, BR, 1), lambda i: (0, i, 0)),
        _vec,
    ],
    out_specs=pl.BlockSpec((BR, D), _blk_rows),
    out_shape=jax.ShapeDtypeStruct((NT, D), _f32),
)


def kernel(x, edge_index, Wc_out, bc_out, Wc_root, Wg1, a_src1, a_dst1,
           bg1, Wg2, a_src2, a_dst2, bg2):
  src, dst = edge_index[0], edge_index[1]
  loop = jnp.arange(N, dtype=_i32)
  pad = EP - E2
  srcp = jnp.concatenate([src, loop, jnp.zeros((pad,), _i32)])
  dstp = jnp.concatenate([dst, loop, jnp.full((pad,), DUMMY, _i32)])
  srcp = srcp.reshape(TILES, C, G)
  dstp = dstp.reshape(TILES, C, G)
  idn = jnp.arange(NT, dtype=_i32).reshape(NG, G)

  xp = jnp.zeros((NT, D), _f32).at[:N].set(x)
  ones_w = jnp.ones((TILES, C, G), _f32)

  wot = Wc_out.T
  wrt = Wc_root.T
  bo = bc_out.reshape(1, D)
  w1t = Wg1.T
  as21 = (Wg1.T @ a_src1).reshape(1, D)
  ad21 = (Wg1.T @ a_dst1).reshape(1, D)
  w2t = Wg2.T
  as22 = (Wg2.T @ a_src2).reshape(1, D)
  ad22 = (Wg2.T @ a_dst2).reshape(1, D)
  b1 = bg1.reshape(1, D)
  b2 = bg2.reshape(1, D)

  k_deg, k_gat_a, k_gat_b, k_spmm = _sc_kernels()

  deg2, dst_r = k_deg(srcp, dstp, idn)
  acc1 = k_spmm(xp, srcp, dst_r, ones_w)
  hw1, sd1 = tc_a(acc1, deg2.reshape(NC, NT, 1), xp,
                  wot, wrt, bo, w1t, as21, ad21)

  e1, m1 = k_gat_a(srcp, dstp, sd1[0], sd1[1])
  ee1, den1 = k_gat_b(dstp, e1, m1, idn)
  acc2 = k_spmm(hw1, srcp, dstp, ee1)
  hw2, sd2 = tc_b(acc2, den1.reshape(NC, NT, 1), b1, w2t, as22, ad22)

  e2, m2 = k_gat_a(srcp, dstp, sd2[0], sd2[1])
  ee2, den2 = k_gat_b(dstp, e2, m2, idn)
  acc3 = k_spmm(hw2, srcp, dstp, ee2)
  out = tc_c(acc3, den2.reshape(NC, NT, 1), b2)

  return out[:N]


# speedup vs baseline: 24.1246x; 1.1967x over previous
import functools

import jax
import jax.numpy as jnp
from jax import lax
from jax.experimental import pallas as pl
from jax.experimental.pallas import tpu as pltpu
from jax.experimental.pallas import tpu_sc as plsc

N = 10000
D = 128
E = 320000
E2 = N + E
NC = 2
NS = 16
TILES = NC * NS
G = 128
C = -(-E2 // (TILES * G))
PT = C * G
EP = TILES * PT
NT = 10240
DUMMY = N
NV = NT // 16
RPT = NT // NS
NG = NT // G
DV = D // 16
BR = 256

_f32 = jnp.float32
_i32 = jnp.int32


def _iota16():
  return lax.iota(_i32, 16)


def _take(v, idx):
  return v.at[idx].get(mode="promise_in_bounds")


def _seg_rmw(tab_ref, keys, vals, *, is_max):
  ks, vs = plsc.sort_key_val(keys, vals)
  ii = _iota16()
  for st in (1, 2, 4, 8):
    idx = jnp.minimum(ii + st, 15)
    nk = _take(ks, idx)
    nv = _take(vs, idx)
    ok = (ii + st <= 15) & (nk == ks)
    comb = jnp.maximum(vs, nv) if is_max else vs + nv
    vs = jnp.where(ok, comb, vs)
  pk = _take(ks, jnp.maximum(ii - 1, 0))
  first = (ks != pk) | (ii == 0)
  old = plsc.load_gather(tab_ref, [ks])
  comb = jnp.maximum(old, vs) if is_max else old + vs
  plsc.store_scatter(tab_ref, [ks], comb, mask=first)


def _fill(ref, nvec, val):
  v = jnp.full((16,), val, _f32)

  def body(i, carry):
    ref[pl.ds(i * 16, 16)] = v
    return carry

  lax.fori_loop(0, nvec, body, 0)


def _combine_sum(tab_ref, shared_ref, idn_ref, zv_ref, out_row, s):
  _fill(zv_ref, RPT // 16, 0.0)
  pltpu.sync_copy(zv_ref, shared_ref.at[pl.ds(s * RPT, RPT)])
  plsc.subcore_barrier()

  def body(q, carry):
    pltpu.sync_copy(tab_ref.at[pl.ds(q * G, G)],
                    shared_ref.at[idn_ref.at[q]], add=True)
    return carry

  lax.fori_loop(0, NG, body, 0)
  plsc.subcore_barrier()
  pltpu.sync_copy(shared_ref.at[pl.ds(s * RPT, RPT)],
                  out_row.at[pl.ds(s * RPT, RPT)])


def _combine_max(tab_ref, stage_ref, red_ref, outsm_ref, out_row, s):
  pltpu.sync_copy(tab_ref, stage_ref.at[s])
  plsc.subcore_barrier()
  for r in range(NS):
    pltpu.sync_copy(stage_ref.at[r, pl.ds(s * RPT, RPT)], red_ref.at[r])

  def body(q, carry):
    acc = red_ref[0, pl.ds(q * 16, 16)]
    for r in range(1, NS):
      acc = jnp.maximum(acc, red_ref[r, pl.ds(q * 16, 16)])
    outsm_ref[pl.ds(q * 16, 16)] = acc
    return carry

  lax.fori_loop(0, RPT // 16, body, 0)
  pltpu.sync_copy(outsm_ref, out_row.at[pl.ds(s * RPT, RPT)])


def _deg_body(srcp, dstp, idn, deg2, dst_r,
              src_v, dst_v, idn_v, tab, zv, shared):
  c = lax.axis_index("c")
  s = lax.axis_index("s")
  t = c * NS + s
  pltpu.sync_copy(srcp.at[t], src_v)
  pltpu.sync_copy(dstp.at[t], dst_v)
  pltpu.sync_copy(idn, idn_v)
  _fill(tab, NV, 0.0)
  base = t * PT

  def chunk(j, carry):
    for k in range(G // 16):
      sv = src_v[j, pl.ds(k * 16, 16)]
      dv = dst_v[j, pl.ds(k * 16, 16)]
      pos = base + j * G + k * 16 + _iota16()
      real = sv != dv
      validf = jnp.where(real | (pos >= E), 1.0, 0.0).astype(_f32)
      _seg_rmw(tab, dv, validf, is_max=False)
      keep = real & (pos < E)
      dst_v[j, pl.ds(k * 16, 16)] = jnp.where(keep, dv, DUMMY)
    return carry

  lax.fori_loop(0, C, chunk, 0)
  pltpu.sync_copy(dst_v, dst_r.at[t])
  _combine_sum(tab, shared, idn_v, zv, deg2.at[c], s)


def _gat_a_body(srcp, dstp, s_hbm, d_hbm, e_out, m2,
                src_v, dst_v, s_tab, d_tab, m_tab, e_v, stage, red, outsm):
  c = lax.axis_index("c")
  s = lax.axis_index("s")
  t = c * NS + s
  pltpu.sync_copy(srcp.at[t], src_v)
  pltpu.sync_copy(dstp.at[t], dst_v)
  pltpu.sync_copy(s_hbm, s_tab)
  pltpu.sync_copy(d_hbm, d_tab)
  _fill(m_tab, NV, -1e30)
  base = t * PT

  def chunk(j, carry):
    for k in range(G // 16):
      sv = src_v[j, pl.ds(k * 16, 16)]
      dv = dst_v[j, pl.ds(k * 16, 16)]
      pos = base + j * G + k * 16 + _iota16()
      ssv = plsc.load_gather(s_tab, [sv])
      ddv = plsc.load_gather(d_tab, [dv])
      z = ssv + ddv
      e16 = jnp.where(z > 0, z, 0.2 * z)
      vb = (sv != dv) | (pos >= E)
      e16 = jnp.where(vb, e16, -1e30)
      e_v[j, pl.ds(k * 16, 16)] = e16
      _seg_rmw(m_tab, dv, e16, is_max=True)
    return carry

  lax.fori_loop(0, C, chunk, 0)
  pltpu.sync_copy(e_v, e_out.at[t])
  _combine_max(m_tab, stage, red, outsm, m2.at[c], s)


def _gat_b_body(dstp, e_in, m2, idn, ee_out, den2,
                dst_v, idn_v, m_tab, tmp_tab, den_tab, e_v, ee_v, zv,
                shared):
  c = lax.axis_index("c")
  s = lax.axis_index("s")
  t = c * NS + s
  pltpu.sync_copy(dstp.at[t], dst_v)
  pltpu.sync_copy(e_in.at[t], e_v)
  pltpu.sync_copy(idn, idn_v)
  pltpu.sync_copy(m2.at[0], m_tab)
  pltpu.sync_copy(m2.at[1], tmp_tab)

  def mcomb(q, carry):
    m_tab[pl.ds(q * 16, 16)] = jnp.maximum(
        m_tab[pl.ds(q * 16, 16)], tmp_tab[pl.ds(q * 16, 16)])
    return carry

  lax.fori_loop(0, NV, mcomb, 0)
  _fill(den_tab, NV, 0.0)

  def chunk(j, carry):
    for k in range(G // 16):
      dv = dst_v[j, pl.ds(k * 16, 16)]
      ev = e_v[j, pl.ds(k * 16, 16)]
      mv = plsc.load_gather(m_tab, [dv])
      eev = jnp.exp(ev - mv)
      ee_v[j, pl.ds(k * 16, 16)] = eev
      _seg_rmw(den_tab, dv, eev, is_max=False)
    return carry

  lax.fori_loop(0, C, chunk, 0)
  pltpu.sync_copy(ee_v, ee_out.at[t])
  _combine_sum(den_tab, shared, idn_v, zv, den2.at[c], s)


def _spmm_body(table, srcp, dstp, w_hbm, acc2,
               dst_v, sb_a, sb_b, wb_a, wb_b, row_a, row_b, acc_sh,
               gsem_a, gsem_b):
  c = lax.axis_index("c")
  s = lax.axis_index("s")
  t = c * NS + s
  pltpu.sync_copy(dstp.at[t], dst_v)

  def zrow(r, carry):
    for v in range(DV):
      row_a[r, pl.ds(v * 16, 16)] = jnp.zeros((16,), _f32)
    return carry

  lax.fori_loop(0, G, zrow, 0)
  for i in range(RPT // G):
    pltpu.sync_copy(row_a, acc_sh.at[pl.ds(s * RPT + i * G, G)])
  plsc.subcore_barrier()

  def stage(j, sb, wb):
    pltpu.sync_copy(srcp.at[t, j], sb)
    pltpu.sync_copy(w_hbm.at[t, j], wb)

  def scale(rowbuf, wb):
    def erow(g, icarry):
      w16 = wb[pl.ds(g * 16, 16)]
      for l in range(16):
        w = w16[l]
        r = g * 16 + l
        for v in range(DV):
          rowbuf[r, pl.ds(v * 16, 16)] = rowbuf[r, pl.ds(v * 16, 16)] * w
      return icarry

    lax.fori_loop(0, G // 16, erow, 0)

  stage(0, sb_a, wb_a)
  pltpu.async_copy(table.at[sb_a], row_a, gsem_a)

  def pair(q, carry):
    j0 = 2 * q
    stage(j0 + 1, sb_b, wb_b)
    pltpu.async_copy(table.at[sb_b], row_b, gsem_b)
    pltpu.make_async_copy(table.at[sb_a], row_a, gsem_a).wait()
    scale(row_a, wb_a)
    pltpu.sync_copy(row_a, acc_sh.at[dst_v.at[j0]], add=True)
    stage(j0 + 2, sb_a, wb_a)
    pltpu.async_copy(table.at[sb_a], row_a, gsem_a)
    pltpu.make_async_copy(table.at[sb_b], row_b, gsem_b).wait()
    scale(row_b, wb_b)
    pltpu.sync_copy(row_b, acc_sh.at[dst_v.at[j0 + 1]], add=True)
    return carry

  lax.fori_loop(0, (C - 1) // 2, pair, 0)
  pltpu.make_async_copy(table.at[sb_a], row_a, gsem_a).wait()
  scale(row_a, wb_a)
  pltpu.sync_copy(row_a, acc_sh.at[dst_v.at[C - 1]], add=True)
  plsc.subcore_barrier()

  for i in range(RPT // G):
    lo = s * RPT + i * G
    pltpu.sync_copy(acc_sh.at[pl.ds(lo, G)], row_a)
    pltpu.sync_copy(row_a, acc2.at[c, pl.ds(lo, G)])


@functools.cache
def _sc_kernels():
  mesh = plsc.VectorSubcoreMesh(
      core_axis_name="c", subcore_axis_name="s",
      num_cores=NC, num_subcores=NS)
  cparams = pltpu.CompilerParams(needs_layout_passes=False)
  k_deg = pl.kernel(
      _deg_body,
      out_type=(
          jax.ShapeDtypeStruct((NC, NT), _f32),
          jax.ShapeDtypeStruct((TILES, C, G), _i32),
      ),
      mesh=mesh,
      compiler_params=cparams,
      scratch_types=[
          pltpu.VMEM((C, G), _i32),
          pltpu.VMEM((C, G), _i32),
          pltpu.VMEM((NG, G), _i32),
          pltpu.VMEM((NT,), _f32),
          pltpu.VMEM((RPT,), _f32),
          pltpu.VMEM_SHARED((NT,), _f32),
      ],
  )
  k_gat_a = pl.kernel(
      _gat_a_body,
      out_type=(
          jax.ShapeDtypeStruct((TILES, C, G), _f32),
          jax.ShapeDtypeStruct((NC, NT), _f32),
      ),
      mesh=mesh,
      compiler_params=cparams,
      scratch_types=[
          pltpu.VMEM((C, G), _i32),
          pltpu.VMEM((C, G), _i32),
          pltpu.VMEM((NT,), _f32),
          pltpu.VMEM((NT,), _f32),
          pltpu.VMEM((NT,), _f32),
          pltpu.VMEM((C, G), _f32),
          pltpu.VMEM_SHARED((NS, NT), _f32),
          pltpu.VMEM((NS, RPT), _f32),
          pltpu.VMEM((RPT,), _f32),
      ],
  )
  k_gat_b = pl.kernel(
      _gat_b_body,
      out_type=(
          jax.ShapeDtypeStruct((TILES, C, G), _f32),
          jax.ShapeDtypeStruct((NC, NT), _f32),
      ),
      mesh=mesh,
      compiler_params=cparams,
      scratch_types=[
          pltpu.VMEM((C, G), _i32),
          pltpu.VMEM((NG, G), _i32),
          pltpu.VMEM((NT,), _f32),
          pltpu.VMEM((NT,), _f32),
          pltpu.VMEM((NT,), _f32),
          pltpu.VMEM((C, G), _f32),
          pltpu.VMEM((C, G), _f32),
          pltpu.VMEM((RPT,), _f32),
          pltpu.VMEM_SHARED((NT,), _f32),
      ],
  )
  k_spmm = pl.kernel(
      _spmm_body,
      out_type=jax.ShapeDtypeStruct((NC, NT, D), _f32),
      mesh=mesh,
      compiler_params=cparams,
      scratch_types=[
          pltpu.VMEM((C, G), _i32),
          pltpu.VMEM((G,), _i32),
          pltpu.VMEM((G,), _i32),
          pltpu.VMEM((G,), _f32),
          pltpu.VMEM((G,), _f32),
          pltpu.VMEM((G, D), _f32),
          pltpu.VMEM((G, D), _f32),
          pltpu.VMEM_SHARED((NT, D), _f32),
          pltpu.SemaphoreType.DMA,
          pltpu.SemaphoreType.DMA,
      ],
  )
  return k_deg, k_gat_a, k_gat_b, k_spmm


def _blk_rows(i):
  return (i, 0)


_full = pl.BlockSpec((D, D), lambda i: (0, 0))
_vec = pl.BlockSpec((1, D), lambda i: (0, 0))


def _tca_body(acc_r, deg_r, x_r, wot_r, wrt_r, bo_r, w1t_r, as2_r, ad2_r,
              hw_r, sd_r):
  dinv = 1.0 / jnp.maximum(deg_r[0] + deg_r[1], 1.0)
  a = (acc_r[0] + acc_r[1] + 2.0 * x_r[...]) * dinv
  h = jnp.dot(a, wot_r[...]) + jnp.dot(x_r[...], wrt_r[...]) + bo_r[0]
  h = jnp.maximum(h, 0.0)
  hw_r[...] = jnp.dot(h, w1t_r[...])
  sd_r[0] = h @ as2_r[0]
  sd_r[1] = h @ ad2_r[0]


tc_a = pl.pallas_call(
    _tca_body,
    grid=(NT // BR,),
    in_specs=[
        pl.BlockSpec((NC, BR, D), lambda i: (0, i, 0)),
        pl.BlockSpec((NC, BR, 1), lambda i: (0, i, 0)),
        pl.BlockSpec((BR, D), _blk_rows),
        _full, _full, _vec, _full, _vec, _vec,
    ],
    out_specs=[
        pl.BlockSpec((BR, D), _blk_rows),
        pl.BlockSpec((2, BR), lambda i: (0, i)),
    ],
    out_shape=[
        jax.ShapeDtypeStruct((NT, D), _f32),
        jax.ShapeDtypeStruct((2, NT), _f32),
    ],
)


def _tcb_body(acc_r, den_r, b_r, w2t_r, as2_r, ad2_r, hw_r, sd_r):
  dscl = 1.0 / (den_r[0] + den_r[1] + 1e-16)
  h = jnp.maximum((acc_r[0] + acc_r[1]) * dscl + b_r[0], 0.0)
  hw_r[...] = jnp.dot(h, w2t_r[...])
  sd_r[0] = h @ as2_r[0]
  sd_r[1] = h @ ad2_r[0]


tc_b = pl.pallas_call(
    _tcb_body,
    grid=(NT // BR,),
    in_specs=[
        pl.BlockSpec((NC, BR, D), lambda i: (0, i, 0)),
        pl.BlockSpec((NC, BR, 1), lambda i: (0, i, 0)),
        _vec, _full, _vec, _vec,
    ],
    out_specs=[
        pl.BlockSpec((BR, D), _blk_rows),
        pl.BlockSpec((2, BR), lambda i: (0, i)),
    ],
    out_shape=[
        jax.ShapeDtypeStruct((NT, D), _f32),
        jax.ShapeDtypeStruct((2, NT), _f32),
    ],
)


def _tcc_body(acc_r, den_r, b_r, out_r):
  dscl = 1.0 / (den_r[0] + den_r[1] + 1e-16)
  out_r[...] = (acc_r[0] + acc_r[1]) * dscl + b_r[0]


tc_c = pl.pallas_call(
    _tcc_body,
    grid=(NT // BR,),
    in_specs=[
        pl.BlockSpec((NC, BR, D), lambda i: (0, i, 0)),
        pl.BlockSpec((NC, BR, 1), lambda i: (0, i, 0)),
        _vec,
    ],
    out_specs=pl.BlockSpec((BR, D), _blk_rows),
    out_shape=jax.ShapeDtypeStruct((NT, D), _f32),
)


def kernel(x, edge_index, Wc_out, bc_out, Wc_root, Wg1, a_src1, a_dst1,
           bg1, Wg2, a_src2, a_dst2, bg2):
  src, dst = edge_index[0], edge_index[1]
  loop = jnp.arange(N, dtype=_i32)
  pad = EP - E2
  srcp = jnp.concatenate([src, loop, jnp.zeros((pad,), _i32)])
  dstp = jnp.concatenate([dst, loop, jnp.full((pad,), DUMMY, _i32)])
  srcp = srcp.reshape(TILES, C, G)
  dstp = dstp.reshape(TILES, C, G)
  idn = jnp.arange(NT, dtype=_i32).reshape(NG, G)

  xp = jnp.zeros((NT, D), _f32).at[:N].set(x)
  ones_w = jnp.ones((TILES, C, G), _f32)

  wot = Wc_out.T
  wrt = Wc_root.T
  bo = bc_out.reshape(1, D)
  w1t = Wg1.T
  as21 = (Wg1.T @ a_src1).reshape(1, D)
  ad21 = (Wg1.T @ a_dst1).reshape(1, D)
  w2t = Wg2.T
  as22 = (Wg2.T @ a_src2).reshape(1, D)
  ad22 = (Wg2.T @ a_dst2).reshape(1, D)
  b1 = bg1.reshape(1, D)
  b2 = bg2.reshape(1, D)

  k_deg, k_gat_a, k_gat_b, k_spmm = _sc_kernels()

  deg2, dst_r = k_deg(srcp, dstp, idn)
  acc1 = k_spmm(xp, srcp, dst_r, ones_w)
  hw1, sd1 = tc_a(acc1, deg2.reshape(NC, NT, 1), xp,
                  wot, wrt, bo, w1t, as21, ad21)

  e1, m1 = k_gat_a(srcp, dstp, sd1[0], sd1[1])
  ee1, den1 = k_gat_b(dstp, e1, m1, idn)
  acc2 = k_spmm(hw1, srcp, dstp, ee1)
  hw2, sd2 = tc_b(acc2, den1.reshape(NC, NT, 1), b1, w2t, as22, ad22)

  e2, m2 = k_gat_a(srcp, dstp, sd2[0], sd2[1])
  ee2, den2 = k_gat_b(dstp, e2, m2, idn)
  acc3 = k_spmm(hw2, srcp, dstp, ee2)
  out = tc_c(acc3, den2.reshape(NC, NT, 1), b2)

  return out[:N]


# speedup vs baseline: 25.6711x; 1.0641x over previous
import functools

import jax
import jax.numpy as jnp
from jax import lax
from jax.experimental import pallas as pl
from jax.experimental.pallas import tpu as pltpu
from jax.experimental.pallas import tpu_sc as plsc

N = 10000
D = 128
E = 320000
E2 = N + E
NC = 2
NS = 16
TILES = NC * NS
G = 96
C = -(-E2 // (TILES * G))
PT = C * G
EP = TILES * PT
NT = 10240
DUMMY = N
NV = NT // 16
RPT = NT // NS
NG = NT // 128
DV = D // 16
BR = 256

_f32 = jnp.float32
_i32 = jnp.int32


def _iota16():
  return lax.iota(_i32, 16)


def _take(v, idx):
  return v.at[idx].get(mode="promise_in_bounds")


def _seg_rmw(tab_ref, keys, vals, *, is_max):
  ks, vs = plsc.sort_key_val(keys, vals)
  ii = _iota16()
  for st in (1, 2, 4, 8):
    idx = jnp.minimum(ii + st, 15)
    nk = _take(ks, idx)
    nv = _take(vs, idx)
    ok = (ii + st <= 15) & (nk == ks)
    comb = jnp.maximum(vs, nv) if is_max else vs + nv
    vs = jnp.where(ok, comb, vs)
  pk = _take(ks, jnp.maximum(ii - 1, 0))
  first = (ks != pk) | (ii == 0)
  old = plsc.load_gather(tab_ref, [ks])
  comb = jnp.maximum(old, vs) if is_max else old + vs
  plsc.store_scatter(tab_ref, [ks], comb, mask=first)


def _fill(ref, nvec, val):
  v = jnp.full((16,), val, _f32)

  def body(i, carry):
    ref[pl.ds(i * 16, 16)] = v
    return carry

  lax.fori_loop(0, nvec, body, 0)


def _combine_sum(tab_ref, shared_ref, idn_ref, zv_ref, out_row, s):
  _fill(zv_ref, RPT // 16, 0.0)
  pltpu.sync_copy(zv_ref, shared_ref.at[pl.ds(s * RPT, RPT)])
  plsc.subcore_barrier()

  def body(q, carry):
    pltpu.sync_copy(tab_ref.at[pl.ds(q * 128, 128)],
                    shared_ref.at[idn_ref.at[q]], add=True)
    return carry

  lax.fori_loop(0, NG, body, 0)
  plsc.subcore_barrier()
  pltpu.sync_copy(shared_ref.at[pl.ds(s * RPT, RPT)],
                  out_row.at[pl.ds(s * RPT, RPT)])


def _combine_max(tab_ref, stage_ref, red_ref, outsm_ref, out_row, s):
  pltpu.sync_copy(tab_ref, stage_ref.at[s])
  plsc.subcore_barrier()
  for r in range(NS):
    pltpu.sync_copy(stage_ref.at[r, pl.ds(s * RPT, RPT)], red_ref.at[r])

  def body(q, carry):
    acc = red_ref[0, pl.ds(q * 16, 16)]
    for r in range(1, NS):
      acc = jnp.maximum(acc, red_ref[r, pl.ds(q * 16, 16)])
    outsm_ref[pl.ds(q * 16, 16)] = acc
    return carry

  lax.fori_loop(0, RPT // 16, body, 0)
  pltpu.sync_copy(outsm_ref, out_row.at[pl.ds(s * RPT, RPT)])


def _deg_body(srcp, dstp, idn, deg2, dst_r,
              src_v, dst_v, idn_v, tab, zv, shared):
  c = lax.axis_index("c")
  s = lax.axis_index("s")
  t = c * NS + s
  pltpu.sync_copy(srcp.at[t], src_v)
  pltpu.sync_copy(dstp.at[t], dst_v)
  pltpu.sync_copy(idn, idn_v)
  _fill(tab, NV, 0.0)
  base = t * PT

  def chunk(j, carry):
    for k in range(G // 16):
      sv = src_v[j, pl.ds(k * 16, 16)]
      dv = dst_v[j, pl.ds(k * 16, 16)]
      pos = base + j * G + k * 16 + _iota16()
      real = sv != dv
      validf = jnp.where(real | (pos >= E), 1.0, 0.0).astype(_f32)
      _seg_rmw(tab, dv, validf, is_max=False)
      keep = real & (pos < E)
      dst_v[j, pl.ds(k * 16, 16)] = jnp.where(keep, dv, DUMMY)
    return carry

  lax.fori_loop(0, C, chunk, 0)
  pltpu.sync_copy(dst_v, dst_r.at[t])
  _combine_sum(tab, shared, idn_v, zv, deg2.at[c], s)


def _gat_a_body(srcp, dstp, s_hbm, d_hbm, e_out, m2,
                src_v, dst_v, s_tab, d_tab, m_tab, e_v, stage, red, outsm):
  c = lax.axis_index("c")
  s = lax.axis_index("s")
  t = c * NS + s
  pltpu.sync_copy(srcp.at[t], src_v)
  pltpu.sync_copy(dstp.at[t], dst_v)
  pltpu.sync_copy(s_hbm, s_tab)
  pltpu.sync_copy(d_hbm, d_tab)
  _fill(m_tab, NV, -1e30)
  base = t * PT

  def chunk(j, carry):
    for k in range(G // 16):
      sv = src_v[j, pl.ds(k * 16, 16)]
      dv = dst_v[j, pl.ds(k * 16, 16)]
      pos = base + j * G + k * 16 + _iota16()
      ssv = plsc.load_gather(s_tab, [sv])
      ddv = plsc.load_gather(d_tab, [dv])
      z = ssv + ddv
      e16 = jnp.where(z > 0, z, 0.2 * z)
      vb = (sv != dv) | (pos >= E)
      e16 = jnp.where(vb, e16, -1e30)
      e_v[j, pl.ds(k * 16, 16)] = e16
      _seg_rmw(m_tab, dv, e16, is_max=True)
    return carry

  lax.fori_loop(0, C, chunk, 0)
  pltpu.sync_copy(e_v, e_out.at[t])
  _combine_max(m_tab, stage, red, outsm, m2.at[c], s)


def _gat_b_body(dstp, e_in, m2, idn, ee_out, den2,
                dst_v, idn_v, m_tab, tmp_tab, den_tab, e_v, ee_v, zv,
                shared):
  c = lax.axis_index("c")
  s = lax.axis_index("s")
  t = c * NS + s
  pltpu.sync_copy(dstp.at[t], dst_v)
  pltpu.sync_copy(e_in.at[t], e_v)
  pltpu.sync_copy(idn, idn_v)
  pltpu.sync_copy(m2.at[0], m_tab)
  pltpu.sync_copy(m2.at[1], tmp_tab)

  def mcomb(q, carry):
    m_tab[pl.ds(q * 16, 16)] = jnp.maximum(
        m_tab[pl.ds(q * 16, 16)], tmp_tab[pl.ds(q * 16, 16)])
    return carry

  lax.fori_loop(0, NV, mcomb, 0)
  _fill(den_tab, NV, 0.0)

  def chunk(j, carry):
    for k in range(G // 16):
      dv = dst_v[j, pl.ds(k * 16, 16)]
      ev = e_v[j, pl.ds(k * 16, 16)]
      mv = plsc.load_gather(m_tab, [dv])
      eev = jnp.exp(ev - mv)
      ee_v[j, pl.ds(k * 16, 16)] = eev
      _seg_rmw(den_tab, dv, eev, is_max=False)
    return carry

  lax.fori_loop(0, C, chunk, 0)
  pltpu.sync_copy(ee_v, ee_out.at[t])
  _combine_sum(den_tab, shared, idn_v, zv, den2.at[c], s)


def _spmm_body(table, esw, acc2, sw0, sw1, sw2, row0, row1, row2, acc_sh,
               gs0, gs1, gs2, ss0, ss1, ss2):
  c = lax.axis_index("c")
  s = lax.axis_index("s")
  t = c * NS + s
  sw = (sw0, sw1, sw2)
  row = (row0, row1, row2)
  gs = (gs0, gs1, gs2)
  ss = (ss0, ss1, ss2)

  def zrow(r, carry):
    for v in range(DV):
      row0[r, pl.ds(v * 16, 16)] = jnp.zeros((16,), _f32)
    return carry

  lax.fori_loop(0, G, zrow, 0)
  zoff = 0
  while zoff < RPT:
    ln = min(G, RPT - zoff)
    pltpu.sync_copy(row0.at[pl.ds(0, ln)],
                    acc_sh.at[pl.ds(s * RPT + zoff, ln)])
    zoff += ln
  plsc.subcore_barrier()

  def stage(j, x):
    pltpu.sync_copy(esw.at[t, j], sw[x])

  def gather(x):
    pltpu.async_copy(table.at[sw[x].at[0]], row[x], gs[x])

  def scale(x):
    rowbuf = row[x]

    def erow(g, icarry):
      w16 = plsc.bitcast(sw[x][1, pl.ds(g * 16, 16)], _f32)
      for l in range(16):
        w = w16[l]
        r = g * 16 + l
        for v in range(DV):
          rowbuf[r, pl.ds(v * 16, 16)] = rowbuf[r, pl.ds(v * 16, 16)] * w
      return icarry

    lax.fori_loop(0, G // 16, erow, 0)

  def scatter(x):
    pltpu.async_copy(row[x], acc_sh.at[sw[x].at[2]], ss[x], add=True)

  def wait_gather(x):
    pltpu.make_async_copy(table.at[sw[x].at[0]], row[x], gs[x]).wait()

  def wait_scatter(x):
    pltpu.make_async_copy(row[x], acc_sh.at[sw[x].at[2]], ss[x]).wait()

  stage(0, 0)
  gather(0)
  stage(1, 1)
  gather(1)
  wait_gather(0)
  scale(0)
  scatter(0)
  stage(2, 2)
  gather(2)

  def trio(q, carry):
    for u in range(3):
      j = 3 * q + 1 + u
      x = (1 + u) % 3
      z = u
      wait_gather(x)
      scale(x)
      scatter(x)
      wait_scatter(z)
      stage(j + 2, z)
      gather(z)
    return carry

  lax.fori_loop(0, (C - 3) // 3, trio, 0)
  for j in (C - 2, C - 1):
    x = j % 3
    wait_gather(x)
    scale(x)
    scatter(x)
  for x in range(3):
    wait_scatter(x)
  plsc.subcore_barrier()

  zoff = 0
  while zoff < RPT:
    ln = min(G, RPT - zoff)
    lo = s * RPT + zoff
    pltpu.sync_copy(acc_sh.at[pl.ds(lo, ln)], row0.at[pl.ds(0, ln)])
    pltpu.sync_copy(row0.at[pl.ds(0, ln)], acc2.at[c, pl.ds(lo, ln)])
    zoff += ln


@functools.cache
def _sc_kernels():
  mesh = plsc.VectorSubcoreMesh(
      core_axis_name="c", subcore_axis_name="s",
      num_cores=NC, num_subcores=NS)
  cparams = pltpu.CompilerParams(needs_layout_passes=False)
  k_deg = pl.kernel(
      _deg_body,
      out_type=(
          jax.ShapeDtypeStruct((NC, NT), _f32),
          jax.ShapeDtypeStruct((TILES, C, G), _i32),
      ),
      mesh=mesh,
      compiler_params=cparams,
      scratch_types=[
          pltpu.VMEM((C, G), _i32),
          pltpu.VMEM((C, G), _i32),
          pltpu.VMEM((NG, 128), _i32),
          pltpu.VMEM((NT,), _f32),
          pltpu.VMEM((RPT,), _f32),
          pltpu.VMEM_SHARED((NT,), _f32),
      ],
  )
  k_gat_a = pl.kernel(
      _gat_a_body,
      out_type=(
          jax.ShapeDtypeStruct((TILES, C, G), _f32),
          jax.ShapeDtypeStruct((NC, NT), _f32),
      ),
      mesh=mesh,
      compiler_params=cparams,
      scratch_types=[
          pltpu.VMEM((C, G), _i32),
          pltpu.VMEM((C, G), _i32),
          pltpu.VMEM((NT,), _f32),
          pltpu.VMEM((NT,), _f32),
          pltpu.VMEM((NT,), _f32),
          pltpu.VMEM((C, G), _f32),
          pltpu.VMEM_SHARED((NS, NT), _f32),
          pltpu.VMEM((NS, RPT), _f32),
          pltpu.VMEM((RPT,), _f32),
      ],
  )
  k_gat_b = pl.kernel(
      _gat_b_body,
      out_type=(
          jax.ShapeDtypeStruct((TILES, C, G), _f32),
          jax.ShapeDtypeStruct((NC, NT), _f32),
      ),
      mesh=mesh,
      compiler_params=cparams,
      scratch_types=[
          pltpu.VMEM((C, G), _i32),
          pltpu.VMEM((NG, 128), _i32),
          pltpu.VMEM((NT,), _f32),
          pltpu.VMEM((NT,), _f32),
          pltpu.VMEM((NT,), _f32),
          pltpu.VMEM((C, G), _f32),
          pltpu.VMEM((C, G), _f32),
          pltpu.VMEM((RPT,), _f32),
          pltpu.VMEM_SHARED((NT,), _f32),
      ],
  )
  k_spmm = pl.kernel(
      _spmm_body,
      out_type=jax.ShapeDtypeStruct((NC, NT, D), _f32),
      mesh=mesh,
      compiler_params=cparams,
      scratch_types=[
          pltpu.VMEM((3, G), _i32),
          pltpu.VMEM((3, G), _i32),
          pltpu.VMEM((3, G), _i32),
          pltpu.VMEM((G, D), _f32),
          pltpu.VMEM((G, D), _f32),
          pltpu.VMEM((G, D), _f32),
          pltpu.VMEM_SHARED((NT, D), _f32),
          pltpu.SemaphoreType.DMA,
          pltpu.SemaphoreType.DMA,
          pltpu.SemaphoreType.DMA,
          pltpu.SemaphoreType.DMA,
          pltpu.SemaphoreType.DMA,
          pltpu.SemaphoreType.DMA,
      ],
  )
  return k_deg, k_gat_a, k_gat_b, k_spmm


def _blk_rows(i):
  return (i, 0)


_full = pl.BlockSpec((D, D), lambda i: (0, 0))
_vec = pl.BlockSpec((1, D), lambda i: (0, 0))


def _tca_body(acc_r, deg_r, x_r, wot_r, wrt_r, bo_r, w1t_r, as2_r, ad2_r,
              hw_r, sd_r):
  dinv = 1.0 / jnp.maximum(deg_r[0] + deg_r[1], 1.0)
  a = (acc_r[0] + acc_r[1] + 2.0 * x_r[...]) * dinv
  h = jnp.dot(a, wot_r[...]) + jnp.dot(x_r[...], wrt_r[...]) + bo_r[0]
  h = jnp.maximum(h, 0.0)
  hw_r[...] = jnp.dot(h, w1t_r[...])
  sd_r[0] = h @ as2_r[0]
  sd_r[1] = h @ ad2_r[0]


tc_a = pl.pallas_call(
    _tca_body,
    grid=(NT // BR,),
    in_specs=[
        pl.BlockSpec((NC, BR, D), lambda i: (0, i, 0)),
        pl.BlockSpec((NC, BR, 1), lambda i: (0, i, 0)),
        pl.BlockSpec((BR, D), _blk_rows),
        _full, _full, _vec, _full, _vec, _vec,
    ],
    out_specs=[
        pl.BlockSpec((BR, D), _blk_rows),
        pl.BlockSpec((2, BR), lambda i: (0, i)),
    ],
    out_shape=[
        jax.ShapeDtypeStruct((NT, D), _f32),
        jax.ShapeDtypeStruct((2, NT), _f32),
    ],
)


def _tcb_body(acc_r, den_r, b_r, w2t_r, as2_r, ad2_r, hw_r, sd_r):
  dscl = 1.0 / (den_r[0] + den_r[1] + 1e-16)
  h = jnp.maximum((acc_r[0] + acc_r[1]) * dscl + b_r[0], 0.0)
  hw_r[...] = jnp.dot(h, w2t_r[...])
  sd_r[0] = h @ as2_r[0]
  sd_r[1] = h @ ad2_r[0]


tc_b = pl.pallas_call(
    _tcb_body,
    grid=(NT // BR,),
    in_specs=[
        pl.BlockSpec((NC, BR, D), lambda i: (0, i, 0)),
        pl.BlockSpec((NC, BR, 1), lambda i: (0, i, 0)),
        _vec, _full, _vec, _vec,
    ],
    out_specs=[
        pl.BlockSpec((BR, D), _blk_rows),
        pl.BlockSpec((2, BR), lambda i: (0, i)),
    ],
    out_shape=[
        jax.ShapeDtypeStruct((NT, D), _f32),
        jax.ShapeDtypeStruct((2, NT), _f32),
    ],
)


def _tcc_body(acc_r, den_r, b_r, out_r):
  dscl = 1.0 / (den_r[0] + den_r[1] + 1e-16)
  out_r[...] = (acc_r[0] + acc_r[1]) * dscl + b_r[0]


tc_c = pl.pallas_call(
    _tcc_body,
    grid=(NT // BR,),
    in_specs=[
        pl.BlockSpec((NC, BR, D), lambda i: (0, i, 0)),
        pl.BlockSpec((NC, BR, 1), lambda i: (0, i, 0)),
        _vec,
    ],
    out_specs=pl.BlockSpec((BR, D), _blk_rows),
    out_shape=jax.ShapeDtypeStruct((NT, D), _f32),
)


def kernel(x, edge_index, Wc_out, bc_out, Wc_root, Wg1, a_src1, a_dst1,
           bg1, Wg2, a_src2, a_dst2, bg2):
  src, dst = edge_index[0], edge_index[1]
  loop = jnp.arange(N, dtype=_i32)
  pad = EP - E2
  srcp = jnp.concatenate([src, loop, jnp.zeros((pad,), _i32)])
  dstp = jnp.concatenate([dst, loop, jnp.full((pad,), DUMMY, _i32)])
  srcp = srcp.reshape(TILES, C, G)
  dstp = dstp.reshape(TILES, C, G)
  idn = jnp.arange(NT, dtype=_i32).reshape(NG, 128)

  xp = jnp.zeros((NT, D), _f32).at[:N].set(x)
  ones_w = jnp.ones((TILES, C, G), _f32)

  wot = Wc_out.T
  wrt = Wc_root.T
  bo = bc_out.reshape(1, D)
  w1t = Wg1.T
  as21 = (Wg1.T @ a_src1).reshape(1, D)
  ad21 = (Wg1.T @ a_dst1).reshape(1, D)
  w2t = Wg2.T
  as22 = (Wg2.T @ a_src2).reshape(1, D)
  ad22 = (Wg2.T @ a_dst2).reshape(1, D)
  b1 = bg1.reshape(1, D)
  b2 = bg2.reshape(1, D)

  k_deg, k_gat_a, k_gat_b, k_spmm = _sc_kernels()

  def pack(srcs, w, dsts):
    wb = lax.bitcast_convert_type(w, _i32)
    return jnp.stack([srcs, wb, dsts], axis=2)

  deg2, dst_r = k_deg(srcp, dstp, idn)
  acc1 = k_spmm(xp, pack(srcp, ones_w, dst_r))
  hw1, sd1 = tc_a(acc1, deg2.reshape(NC, NT, 1), xp,
                  wot, wrt, bo, w1t, as21, ad21)

  e1, m1 = k_gat_a(srcp, dstp, sd1[0], sd1[1])
  ee1, den1 = k_gat_b(dstp, e1, m1, idn)
  acc2 = k_spmm(hw1, pack(srcp, ee1, dstp))
  hw2, sd2 = tc_b(acc2, den1.reshape(NC, NT, 1), b1, w2t, as22, ad22)

  e2, m2 = k_gat_a(srcp, dstp, sd2[0], sd2[1])
  ee2, den2 = k_gat_b(dstp, e2, m2, idn)
  acc3 = k_spmm(hw2, pack(srcp, ee2, dstp))
  out = tc_c(acc3, den2.reshape(NC, NT, 1), b2)

  return out[:N]


# speedup vs baseline: 27.6444x; 1.0769x over previous
import functools

import jax
import jax.numpy as jnp
from jax import lax
from jax.experimental import pallas as pl
from jax.experimental.pallas import tpu as pltpu
from jax.experimental.pallas import tpu_sc as plsc

N = 10000
D = 128
E = 320000
E2 = N + E
NC = 2
NS = 16
TILES = NC * NS
G = 96
C = -(-E2 // (TILES * G))
CH0 = 132
CH1 = 2 * C - CH0
EP = NS * G * (CH0 + CH1)
NT = 10240
DUMMY = N
NV = NT // 16
RPT = NT // NS
NG = NT // 128
DV = D // 16
BR = 256

_f32 = jnp.float32
_i32 = jnp.int32


def _iota16():
  return lax.iota(_i32, 16)


def _take(v, idx):
  return v.at[idx].get(mode="promise_in_bounds")


def _seg_rmw(tab_ref, keys, vals, *, is_max):
  ks, vs = plsc.sort_key_val(keys, vals)
  ii = _iota16()
  for st in (1, 2, 4, 8):
    idx = jnp.minimum(ii + st, 15)
    nk = _take(ks, idx)
    nv = _take(vs, idx)
    ok = (ii + st <= 15) & (nk == ks)
    comb = jnp.maximum(vs, nv) if is_max else vs + nv
    vs = jnp.where(ok, comb, vs)
  pk = _take(ks, jnp.maximum(ii - 1, 0))
  first = (ks != pk) | (ii == 0)
  old = plsc.load_gather(tab_ref, [ks])
  comb = jnp.maximum(old, vs) if is_max else old + vs
  plsc.store_scatter(tab_ref, [ks], comb, mask=first)


def _fill(ref, nvec, val):
  v = jnp.full((16,), val, _f32)

  def body(i, carry):
    ref[pl.ds(i * 16, 16)] = v
    return carry

  lax.fori_loop(0, nvec, body, 0)


def _combine_sum(tab_ref, shared_ref, idn_ref, zv_ref, out_row, s):
  _fill(zv_ref, RPT // 16, 0.0)
  pltpu.sync_copy(zv_ref, shared_ref.at[pl.ds(s * RPT, RPT)])
  plsc.subcore_barrier()

  def body(q, carry):
    pltpu.sync_copy(tab_ref.at[pl.ds(q * 128, 128)],
                    shared_ref.at[idn_ref.at[q]], add=True)
    return carry

  lax.fori_loop(0, NG, body, 0)
  plsc.subcore_barrier()
  pltpu.sync_copy(shared_ref.at[pl.ds(s * RPT, RPT)],
                  out_row.at[pl.ds(s * RPT, RPT)])


def _combine_max(tab_ref, stage_ref, red_ref, outsm_ref, out_row, s):
  pltpu.sync_copy(tab_ref, stage_ref.at[s])
  plsc.subcore_barrier()
  for r in range(NS):
    pltpu.sync_copy(stage_ref.at[r, pl.ds(s * RPT, RPT)], red_ref.at[r])

  def body(q, carry):
    acc = red_ref[0, pl.ds(q * 16, 16)]
    for r in range(1, NS):
      acc = jnp.maximum(acc, red_ref[r, pl.ds(q * 16, 16)])
    outsm_ref[pl.ds(q * 16, 16)] = acc
    return carry

  lax.fori_loop(0, RPT // 16, body, 0)
  pltpu.sync_copy(outsm_ref, out_row.at[pl.ds(s * RPT, RPT)])


def _core_layout(c, s):
  t = c * NS + s
  nch = jnp.where(c == 0, CH0, CH1)
  base = jnp.where(c == 0, t * (CH0 * G), NS * CH0 * G + s * (CH1 * G))
  return t, nch, base


def _deg_body(srcp, dstp, idn, deg2, dst_r,
              src_v, dst_v, idn_v, tab, zv, shared):
  c = lax.axis_index("c")
  s = lax.axis_index("s")
  t, nch, base = _core_layout(c, s)
  pltpu.sync_copy(srcp.at[t], src_v)
  pltpu.sync_copy(dstp.at[t], dst_v)
  pltpu.sync_copy(idn, idn_v)
  _fill(tab, NV, 0.0)

  def chunk(j, carry):
    for k in range(G // 16):
      sv = src_v[j, pl.ds(k * 16, 16)]
      dv = dst_v[j, pl.ds(k * 16, 16)]
      pos = base + j * G + k * 16 + _iota16()
      real = sv != dv
      validf = jnp.where(real | (pos >= E), 1.0, 0.0).astype(_f32)
      _seg_rmw(tab, dv, validf, is_max=False)
      keep = real & (pos < E)
      dst_v[j, pl.ds(k * 16, 16)] = jnp.where(keep, dv, DUMMY)
    return carry

  lax.fori_loop(0, nch, chunk, 0)
  pltpu.sync_copy(dst_v, dst_r.at[t])
  _combine_sum(tab, shared, idn_v, zv, deg2.at[c], s)


def _gat_a_body(srcp, dstp, s_hbm, d_hbm, e_out, m2,
                src_v, dst_v, s_tab, d_tab, m_tab, e_v, stage, red, outsm):
  c = lax.axis_index("c")
  s = lax.axis_index("s")
  t, nch, base = _core_layout(c, s)
  pltpu.sync_copy(srcp.at[t], src_v)
  pltpu.sync_copy(dstp.at[t], dst_v)
  pltpu.sync_copy(s_hbm, s_tab)
  pltpu.sync_copy(d_hbm, d_tab)
  _fill(m_tab, NV, -1e30)

  def chunk(j, carry):
    for k in range(G // 16):
      sv = src_v[j, pl.ds(k * 16, 16)]
      dv = dst_v[j, pl.ds(k * 16, 16)]
      pos = base + j * G + k * 16 + _iota16()
      ssv = plsc.load_gather(s_tab, [sv])
      ddv = plsc.load_gather(d_tab, [dv])
      z = ssv + ddv
      e16 = jnp.where(z > 0, z, 0.2 * z)
      vb = (sv != dv) | (pos >= E)
      e16 = jnp.where(vb, e16, -1e30)
      e_v[j, pl.ds(k * 16, 16)] = e16
      _seg_rmw(m_tab, dv, e16, is_max=True)
    return carry

  lax.fori_loop(0, nch, chunk, 0)
  pltpu.sync_copy(e_v, e_out.at[t])
  _combine_max(m_tab, stage, red, outsm, m2.at[c], s)


def _gat_b_body(dstp, e_in, m2, idn, ee_out, den2,
                dst_v, idn_v, m_tab, tmp_tab, den_tab, e_v, ee_v, zv,
                shared):
  c = lax.axis_index("c")
  s = lax.axis_index("s")
  t, nch, _ = _core_layout(c, s)
  pltpu.sync_copy(dstp.at[t], dst_v)
  pltpu.sync_copy(e_in.at[t], e_v)
  pltpu.sync_copy(idn, idn_v)
  pltpu.sync_copy(m2.at[0], m_tab)
  pltpu.sync_copy(m2.at[1], tmp_tab)

  def mcomb(q, carry):
    m_tab[pl.ds(q * 16, 16)] = jnp.maximum(
        m_tab[pl.ds(q * 16, 16)], tmp_tab[pl.ds(q * 16, 16)])
    return carry

  lax.fori_loop(0, NV, mcomb, 0)
  _fill(den_tab, NV, 0.0)

  def chunk(j, carry):
    for k in range(G // 16):
      dv = dst_v[j, pl.ds(k * 16, 16)]
      ev = e_v[j, pl.ds(k * 16, 16)]
      mv = plsc.load_gather(m_tab, [dv])
      eev = jnp.exp(ev - mv)
      ee_v[j, pl.ds(k * 16, 16)] = eev
      _seg_rmw(den_tab, dv, eev, is_max=False)
    return carry

  lax.fori_loop(0, nch, chunk, 0)
  pltpu.sync_copy(ee_v, ee_out.at[t])
  _combine_sum(den_tab, shared, idn_v, zv, den2.at[c], s)


def _spmm_body(table, esw, acc2, sw0, sw1, sw2, row0, row1, row2, acc_sh,
               gs0, gs1, gs2, ss0, ss1, ss2):
  c = lax.axis_index("c")
  s = lax.axis_index("s")
  t, nch, _ = _core_layout(c, s)
  sw = (sw0, sw1, sw2)
  row = (row0, row1, row2)
  gs = (gs0, gs1, gs2)
  ss = (ss0, ss1, ss2)

  def zrow(r, carry):
    for v in range(DV):
      row0[r, pl.ds(v * 16, 16)] = jnp.zeros((16,), _f32)
    return carry

  lax.fori_loop(0, G, zrow, 0)
  zoff = 0
  while zoff < RPT:
    ln = min(G, RPT - zoff)
    pltpu.sync_copy(row0.at[pl.ds(0, ln)],
                    acc_sh.at[pl.ds(s * RPT + zoff, ln)])
    zoff += ln
  plsc.subcore_barrier()

  def stage(j, x):
    pltpu.sync_copy(esw.at[t, j], sw[x])

  def gather(x):
    pltpu.async_copy(table.at[sw[x].at[0]], row[x], gs[x])

  def scale(x):
    rowbuf = row[x]

    def erow(g, icarry):
      w16 = plsc.bitcast(sw[x][1, pl.ds(g * 16, 16)], _f32)
      for l in range(16):
        w = w16[l]
        r = g * 16 + l
        for v in range(DV):
          rowbuf[r, pl.ds(v * 16, 16)] = rowbuf[r, pl.ds(v * 16, 16)] * w
      return icarry

    lax.fori_loop(0, G // 16, erow, 0)

  def scatter(x):
    pltpu.async_copy(row[x], acc_sh.at[sw[x].at[2]], ss[x], add=True)

  def wait_gather(x):
    pltpu.make_async_copy(table.at[sw[x].at[0]], row[x], gs[x]).wait()

  def wait_scatter(x):
    pltpu.make_async_copy(row[x], acc_sh.at[sw[x].at[2]], ss[x]).wait()

  stage(0, 0)
  gather(0)
  stage(1, 1)
  gather(1)
  wait_gather(0)
  scale(0)
  scatter(0)
  stage(2, 2)
  gather(2)

  def trio(q, carry):
    for u in range(3):
      j = 3 * q + 1 + u
      x = (1 + u) % 3
      z = u
      wait_gather(x)
      scale(x)
      scatter(x)
      wait_scatter(z)
      stage(j + 2, z)
      gather(z)
    return carry

  lax.fori_loop(0, nch // 3 - 1, trio, 0)
  for j, x in ((nch - 2, 1), (nch - 1, 2)):
    wait_gather(x)
    scale(x)
    scatter(x)
  for x in range(3):
    wait_scatter(x)
  plsc.subcore_barrier()

  zoff = 0
  while zoff < RPT:
    ln = min(G, RPT - zoff)
    lo = s * RPT + zoff
    pltpu.sync_copy(acc_sh.at[pl.ds(lo, ln)], row0.at[pl.ds(0, ln)])
    pltpu.sync_copy(row0.at[pl.ds(0, ln)], acc2.at[c, pl.ds(lo, ln)])
    zoff += ln


@functools.cache
def _sc_kernels():
  mesh = plsc.VectorSubcoreMesh(
      core_axis_name="c", subcore_axis_name="s",
      num_cores=NC, num_subcores=NS)
  cparams = pltpu.CompilerParams(needs_layout_passes=False)
  k_deg = pl.kernel(
      _deg_body,
      out_type=(
          jax.ShapeDtypeStruct((NC, NT), _f32),
          jax.ShapeDtypeStruct((TILES, CH0, G), _i32),
      ),
      mesh=mesh,
      compiler_params=cparams,
      scratch_types=[
          pltpu.VMEM((CH0, G), _i32),
          pltpu.VMEM((CH0, G), _i32),
          pltpu.VMEM((NG, 128), _i32),
          pltpu.VMEM((NT,), _f32),
          pltpu.VMEM((RPT,), _f32),
          pltpu.VMEM_SHARED((NT,), _f32),
      ],
  )
  k_gat_a = pl.kernel(
      _gat_a_body,
      out_type=(
          jax.ShapeDtypeStruct((TILES, CH0, G), _f32),
          jax.ShapeDtypeStruct((NC, NT), _f32),
      ),
      mesh=mesh,
      compiler_params=cparams,
      scratch_types=[
          pltpu.VMEM((CH0, G), _i32),
          pltpu.VMEM((CH0, G), _i32),
          pltpu.VMEM((NT,), _f32),
          pltpu.VMEM((NT,), _f32),
          pltpu.VMEM((NT,), _f32),
          pltpu.VMEM((CH0, G), _f32),
          pltpu.VMEM_SHARED((NS, NT), _f32),
          pltpu.VMEM((NS, RPT), _f32),
          pltpu.VMEM((RPT,), _f32),
      ],
  )
  k_gat_b = pl.kernel(
      _gat_b_body,
      out_type=(
          jax.ShapeDtypeStruct((TILES, CH0, G), _f32),
          jax.ShapeDtypeStruct((NC, NT), _f32),
      ),
      mesh=mesh,
      compiler_params=cparams,
      scratch_types=[
          pltpu.VMEM((CH0, G), _i32),
          pltpu.VMEM((NG, 128), _i32),
          pltpu.VMEM((NT,), _f32),
          pltpu.VMEM((NT,), _f32),
          pltpu.VMEM((NT,), _f32),
          pltpu.VMEM((CH0, G), _f32),
          pltpu.VMEM((CH0, G), _f32),
          pltpu.VMEM((RPT,), _f32),
          pltpu.VMEM_SHARED((NT,), _f32),
      ],
  )
  k_spmm = pl.kernel(
      _spmm_body,
      out_type=jax.ShapeDtypeStruct((NC, NT, D), _f32),
      mesh=mesh,
      compiler_params=cparams,
      scratch_types=[
          pltpu.VMEM((3, G), _i32),
          pltpu.VMEM((3, G), _i32),
          pltpu.VMEM((3, G), _i32),
          pltpu.VMEM((G, D), _f32),
          pltpu.VMEM((G, D), _f32),
          pltpu.VMEM((G, D), _f32),
          pltpu.VMEM_SHARED((NT, D), _f32),
          pltpu.SemaphoreType.DMA,
          pltpu.SemaphoreType.DMA,
          pltpu.SemaphoreType.DMA,
          pltpu.SemaphoreType.DMA,
          pltpu.SemaphoreType.DMA,
          pltpu.SemaphoreType.DMA,
      ],
  )
  return k_deg, k_gat_a, k_gat_b, k_spmm


def _blk_rows(i):
  return (i, 0)


_full = pl.BlockSpec((D, D), lambda i: (0, 0))
_vec = pl.BlockSpec((1, D), lambda i: (0, 0))


def _tca_body(acc_r, deg_r, x_r, wot_r, wrt_r, bo_r, w1t_r, as2_r, ad2_r,
              hw_r, sd_r):
  dinv = 1.0 / jnp.maximum(deg_r[0] + deg_r[1], 1.0)
  a = (acc_r[0] + acc_r[1] + 2.0 * x_r[...]) * dinv
  h = jnp.dot(a, wot_r[...]) + jnp.dot(x_r[...], wrt_r[...]) + bo_r[0]
  h = jnp.maximum(h, 0.0)
  hw_r[...] = jnp.dot(h, w1t_r[...])
  sd_r[0] = h @ as2_r[0]
  sd_r[1] = h @ ad2_r[0]


tc_a = pl.pallas_call(
    _tca_body,
    grid=(NT // BR,),
    in_specs=[
        pl.BlockSpec((NC, BR, D), lambda i: (0, i, 0)),
        pl.BlockSpec((NC, BR, 1), lambda i: (0, i, 0)),
        pl.BlockSpec((BR, D), _blk_rows),
        _full, _full, _vec, _full, _vec, _vec,
    ],
    out_specs=[
        pl.BlockSpec((BR, D), _blk_rows),
        pl.BlockSpec((2, BR), lambda i: (0, i)),
    ],
    out_shape=[
        jax.ShapeDtypeStruct((NT, D), _f32),
        jax.ShapeDtypeStruct((2, NT), _f32),
    ],
)


def _tcb_body(acc_r, den_r, b_r, w2t_r, as2_r, ad2_r, hw_r, sd_r):
  dscl = 1.0 / (den_r[0] + den_r[1] + 1e-16)
  h = jnp.maximum((acc_r[0] + acc_r[1]) * dscl + b_r[0], 0.0)
  hw_r[...] = jnp.dot(h, w2t_r[...])
  sd_r[0] = h @ as2_r[0]
  sd_r[1] = h @ ad2_r[0]


tc_b = pl.pallas_call(
    _tcb_body,
    grid=(NT // BR,),
    in_specs=[
        pl.BlockSpec((NC, BR, D), lambda i: (0, i, 0)),
        pl.BlockSpec((NC, BR, 1), lambda i: (0, i, 0)),
        _vec, _full, _vec, _vec,
    ],
    out_specs=[
        pl.BlockSpec((BR, D), _blk_rows),
        pl.BlockSpec((2, BR), lambda i: (0, i)),
    ],
    out_shape=[
        jax.ShapeDtypeStruct((NT, D), _f32),
        jax.ShapeDtypeStruct((2, NT), _f32),
    ],
)


def _tcc_body(acc_r, den_r, b_r, out_r):
  dscl = 1.0 / (den_r[0] + den_r[1] + 1e-16)
  out_r[...] = (acc_r[0] + acc_r[1]) * dscl + b_r[0]


tc_c = pl.pallas_call(
    _tcc_body,
    grid=(NT // BR,),
    in_specs=[
        pl.BlockSpec((NC, BR, D), lambda i: (0, i, 0)),
        pl.BlockSpec((NC, BR, 1), lambda i: (0, i, 0)),
        _vec,
    ],
    out_specs=pl.BlockSpec((BR, D), _blk_rows),
    out_shape=jax.ShapeDtypeStruct((NT, D), _f32),
)


def kernel(x, edge_index, Wc_out, bc_out, Wc_root, Wg1, a_src1, a_dst1,
           bg1, Wg2, a_src2, a_dst2, bg2):
  src, dst = edge_index[0], edge_index[1]
  loop = jnp.arange(N, dtype=_i32)
  pad = EP - E2

  def repack(flat, padval):
    h0 = flat[:NS * CH0 * G].reshape(NS, CH0, G)
    h1 = flat[NS * CH0 * G:].reshape(NS, CH1, G)
    h1 = jnp.pad(h1, ((0, 0), (0, CH0 - CH1), (0, 0)),
                 constant_values=padval)
    return jnp.concatenate([h0, h1], 0)

  srcp = repack(jnp.concatenate([src, loop, jnp.zeros((pad,), _i32)]), 0)
  dstp = repack(jnp.concatenate([dst, loop, jnp.full((pad,), DUMMY, _i32)]),
                DUMMY)
  idn = jnp.arange(NT, dtype=_i32).reshape(NG, 128)

  xp = jnp.zeros((NT, D), _f32).at[:N].set(x)
  ones_w = jnp.ones((TILES, CH0, G), _f32)

  wot = Wc_out.T
  wrt = Wc_root.T
  bo = bc_out.reshape(1, D)
  w1t = Wg1.T
  as21 = (Wg1.T @ a_src1).reshape(1, D)
  ad21 = (Wg1.T @ a_dst1).reshape(1, D)
  w2t = Wg2.T
  as22 = (Wg2.T @ a_src2).reshape(1, D)
  ad22 = (Wg2.T @ a_dst2).reshape(1, D)
  b1 = bg1.reshape(1, D)
  b2 = bg2.reshape(1, D)

  k_deg, k_gat_a, k_gat_b, k_spmm = _sc_kernels()

  def pack(srcs, w, dsts):
    wb = lax.bitcast_convert_type(w, _i32)
    return jnp.stack([srcs, wb, dsts], axis=2)

  deg2, dst_r = k_deg(srcp, dstp, idn)
  acc1 = k_spmm(xp, pack(srcp, ones_w, dst_r))
  hw1, sd1 = tc_a(acc1, deg2.reshape(NC, NT, 1), xp,
                  wot, wrt, bo, w1t, as21, ad21)

  e1, m1 = k_gat_a(srcp, dstp, sd1[0], sd1[1])
  ee1, den1 = k_gat_b(dstp, e1, m1, idn)
  acc2 = k_spmm(hw1, pack(srcp, ee1, dstp))
  hw2, sd2 = tc_b(acc2, den1.reshape(NC, NT, 1), b1, w2t, as22, ad22)

  e2, m2 = k_gat_a(srcp, dstp, sd2[0], sd2[1])
  ee2, den2 = k_gat_b(dstp, e2, m2, idn)
  acc3 = k_spmm(hw2, pack(srcp, ee2, dstp))
  out = tc_c(acc3, den2.reshape(NC, NT, 1), b2)

  return out[:N]
